# baseline (speedup 1.0000x reference)
import functools
import math

import jax
import jax.numpy as jnp
from jax import lax
from jax.experimental import pallas as pl
from jax.experimental.pallas import tpu as pltpu
from jax.experimental.pallas import tpu_sc as plsc

_LANES = 16
_NBUF = 4
_CS = 4
_COLS_UNROLL = 8


def kernel(x, token_table, pos_table):
    B, S = x.shape
    V, D = token_table.shape
    T = B * S
    scale = math.sqrt(float(D))

    info = plsc.get_sparse_core_info()
    NW = info.num_cores * info.num_subcores
    SW = S // NW
    n_chunks = SW // _CS
    CR = _CS * B
    n_cc = D // (_LANES * _COLS_UNROLL)

    idx = (
        x.astype(jnp.int32)
        .reshape(B, NW, n_chunks, _CS)
        .transpose(1, 2, 0, 3)
        .reshape(NW, n_chunks, CR)
    )
    mesh = plsc.VectorSubcoreMesh(core_axis_name="c", subcore_axis_name="s")

    @functools.partial(
        pl.kernel,
        mesh=mesh,
        out_type=jax.ShapeDtypeStruct((T, D), jnp.float32),
        scratch_types=[
            pltpu.VMEM((n_chunks, CR), jnp.int32),
            pltpu.VMEM((SW, D), jnp.float32),
        ]
        + [pltpu.VMEM((CR, D), jnp.float32) for _ in range(_NBUF)]
        + [pltpu.SemaphoreType.DMA for _ in range(2 * _NBUF + 1)],
    )
    def emb_kernel(x_hbm, tok_hbm, pos_hbm, out_hbm, idx_v, pos_v, *rest):
        tok_bufs = rest[:_NBUF]
        gsem = rest[_NBUF : 2 * _NBUF]
        wsem = rest[2 * _NBUF : 3 * _NBUF]
        psem = rest[3 * _NBUF]
        wid = lax.axis_index("s") * info.num_cores + lax.axis_index("c")
        s0 = wid * SW
        scale_v = jnp.full((_LANES,), scale, jnp.float32)

        pltpu.sync_copy(x_hbm.at[wid], idx_v)
        ph = pltpu.async_copy(pos_hbm.at[pl.ds(s0, SW)], pos_v, psem)
        gh = [None] * n_chunks
        wh = {}
        for p in range(min(2, n_chunks)):
            gh[p] = pltpu.async_copy(
                tok_hbm.at[idx_v.at[p]], tok_bufs[p % _NBUF], gsem[p % _NBUF]
            )
        ph.wait()

        for j in range(n_chunks):
            buf = j % _NBUF
            jn = j + 2
            if jn < n_chunks:
                bn = jn % _NBUF
                if jn - _NBUF >= 0:
                    for h in wh[jn - _NBUF]:
                        h.wait()
                gh[jn] = pltpu.async_copy(
                    tok_hbm.at[idx_v.at[jn]], tok_bufs[bn], gsem[bn]
                )
            gh[j].wait()

            tb = tok_bufs[buf]

            def si_body(si, carry, tb=tb, j=j):
                def cc_body(cc, carry2):
                    cbase = cc * (_LANES * _COLS_UNROLL)
                    for u in range(_COLS_UNROLL):
                        sl = pl.ds(cbase + u * _LANES, _LANES)
                        p = pos_v[j * _CS + si, sl]
                        for b in range(B):
                            r = b * _CS + si
                            tb[r, sl] = tb[r, sl] * scale_v + p
                    return carry2

                lax.fori_loop(0, n_cc, cc_body, 0)
                return carry

            lax.fori_loop(0, _CS, si_body, 0)

            wh[j] = [
                pltpu.async_copy(
                    tb.at[pl.ds(b * _CS, _CS)],
                    out_hbm.at[pl.ds(b * S + s0 + j * _CS, _CS)],
                    wsem[buf],
                )
                for b in range(B)
            ]
        for j in range(max(0, n_chunks - _NBUF), n_chunks):
            for h in wh[j]:
                h.wait()

    out = emb_kernel(idx, token_table, pos_table)
    return out.reshape(B, S, D)


# speedup vs baseline: 1.5342x; 1.5342x over previous
import functools
import math

import jax
import jax.numpy as jnp
from jax import lax
from jax.experimental import pallas as pl
from jax.experimental.pallas import tpu as pltpu
from jax.experimental.pallas import tpu_sc as plsc

_LANES = 16
_NBUF = 4
_CS = 8
_LOOKAHEAD = 2


def kernel(x, token_table, pos_table):
    B, S = x.shape
    V, D = token_table.shape
    T = B * S
    scale = math.sqrt(float(D))

    info = plsc.get_sparse_core_info()
    NW = info.num_cores * info.num_subcores
    SW = S // NW
    n_chunks = SW // _CS
    CR = _CS * B

    idx = (
        x.astype(jnp.int32)
        .reshape(B, NW, n_chunks, _CS)
        .transpose(1, 2, 0, 3)
        .reshape(NW, n_chunks, CR)
    )
    mesh = plsc.VectorSubcoreMesh(core_axis_name="c", subcore_axis_name="s")

    @functools.partial(
        pl.kernel,
        mesh=mesh,
        out_type=jax.ShapeDtypeStruct((T, D), jnp.float32),
        scratch_types=[pltpu.VMEM((n_chunks, CR), jnp.int32)]
        + [pltpu.VMEM((CR, D), jnp.float32) for _ in range(_NBUF)]
        + [pltpu.VMEM((_CS, D), jnp.float32) for _ in range(_NBUF)]
        + [pltpu.SemaphoreType.DMA for _ in range(3 * _NBUF)],
    )
    def emb_kernel(x_hbm, tok_hbm, pos_hbm, out_hbm, idx_v, *rest):
        tok_bufs = rest[:_NBUF]
        pos_bufs = rest[_NBUF : 2 * _NBUF]
        gsem = rest[2 * _NBUF : 3 * _NBUF]
        psem = rest[3 * _NBUF : 4 * _NBUF]
        wsem = rest[4 * _NBUF : 5 * _NBUF]
        wid = lax.axis_index("s") * info.num_cores + lax.axis_index("c")
        s0 = wid * SW
        scale_v = jnp.full((_LANES,), scale, jnp.float32)

        pltpu.sync_copy(x_hbm.at[wid], idx_v)

        def issue(j):
            b = j % _NBUF
            g = pltpu.async_copy(tok_hbm.at[idx_v.at[j]], tok_bufs[b], gsem[b])
            p = pltpu.async_copy(
                pos_hbm.at[pl.ds(s0 + j * _CS, _CS)], pos_bufs[b], psem[b]
            )
            return (g, p)

        gh = {p: issue(p) for p in range(min(_LOOKAHEAD, n_chunks))}
        wh = {}
        for j in range(n_chunks):
            buf = j % _NBUF
            jn = j + _LOOKAHEAD
            if jn < n_chunks:
                if jn - _NBUF >= 0:
                    for h in wh.pop(jn - _NBUF):
                        h.wait()
                gh[jn] = issue(jn)
            for h in gh.pop(j):
                h.wait()

            tb = tok_bufs[buf]
            pb = pos_bufs[buf]

            def si_body(si, carry, tb=tb, pb=pb):
                for c in range(D // _LANES):
                    sl = pl.ds(c * _LANES, _LANES)
                    p = pb[si, sl]
                    for b in range(B):
                        r = b * _CS + si
                        tb[r, sl] = tb[r, sl] * scale_v + p
                return carry

            lax.fori_loop(0, _CS, si_body, 0)

            wh[j] = [
                pltpu.async_copy(
                    tb.at[pl.ds(b * _CS, _CS)],
                    out_hbm.at[pl.ds(b * S + s0 + j * _CS, _CS)],
                    wsem[buf],
                )
                for b in range(B)
            ]
        for j in sorted(wh):
            for h in wh[j]:
                h.wait()

    out = emb_kernel(idx, token_table, pos_table)
    return out.reshape(B, S, D)
